# baseline (speedup 1.0000x reference)
import functools

import jax
import jax.numpy as jnp
from jax import lax
from jax.experimental import pallas as pl
from jax.experimental.pallas import tpu as pltpu
from jax.experimental.pallas import tpu_sc as plsc

D = 64
SEQ = 200
L = 16
NC = 2
NS = 16
NW = NC * NS

CHUNK = 100
NBUF = 4
SCALE = 8.0


def _sc_body(idx_hbm, pos_hbm, tab_hbm, out_hbm, idx_v, pos_v, bufs, sems,
             *, nchunk, per_w):
    wid = lax.axis_index("s") * NC + lax.axis_index("c")
    pltpu.sync_copy(idx_hbm.at[wid], idx_v)
    pltpu.sync_copy(pos_hbm, pos_v)
    for b in range(NBUF):
        pltpu.async_copy(tab_hbm.at[idx_v.at[b]], bufs[b], sems[b])
    row0 = wid * per_w

    def outer(it, carry):
        g0 = it * NBUF
        for b in range(NBUF):
            g = g0 + b
            buf, sem = bufs[b], sems[b]
            pltpu.make_async_copy(tab_hbm.at[idx_v.at[g]], buf, sem).wait()
            poff = (b % 2) * CHUNK

            def ew(rr, c2, buf=buf, poff=poff):
                r = rr * 4
                for j in range(4):
                    for c in range(D // L):
                        sl = pl.ds(c * L, L)
                        buf[r + j, sl] = (buf[r + j, sl] * SCALE
                                          + pos_v[poff + r + j, sl])
                return c2
            lax.fori_loop(0, CHUNK // 4, ew, 0, unroll=False)
            pltpu.sync_copy(buf, out_hbm.at[pl.ds(row0 + g * CHUNK, CHUNK)])
            nxt = g + NBUF

            @pl.when(nxt < nchunk)
            def _():
                pltpu.async_copy(tab_hbm.at[idx_v.at[nxt]], buf, sem)
        return carry

    lax.fori_loop(0, nchunk // NBUF, outer, 0, unroll=False)


@functools.partial(jax.jit, static_argnames=())
def kernel(inputs, token_table, position_table):
    bsz, seq = inputs.shape
    vocab, d = token_table.shape
    total = bsz * seq
    per_w = total // NW
    nchunk = per_w // CHUNK

    idx = inputs.reshape(NW, nchunk, CHUNK).astype(jnp.int32)

    mesh = plsc.VectorSubcoreMesh(core_axis_name="c", subcore_axis_name="s")
    body = functools.partial(_sc_body, nchunk=nchunk, per_w=per_w)

    def wrapped(idx_hbm, pos_hbm, tab_hbm, out_hbm,
                idx_v, pos_v, b0, b1, b2, b3, s0, s1, s2, s3):
        body(idx_hbm, pos_hbm, tab_hbm, out_hbm, idx_v, pos_v,
             (b0, b1, b2, b3), (s0, s1, s2, s3))

    out = pl.kernel(
        wrapped,
        out_type=jax.ShapeDtypeStruct((total, d), jnp.float32),
        mesh=mesh,
        compiler_params=pltpu.CompilerParams(use_tc_tiling_on_sc=False),
        scratch_types=[
            pltpu.VMEM((nchunk, CHUNK), jnp.int32),
            pltpu.VMEM((seq, d), jnp.float32),
            pltpu.VMEM((CHUNK, d), jnp.float32),
            pltpu.VMEM((CHUNK, d), jnp.float32),
            pltpu.VMEM((CHUNK, d), jnp.float32),
            pltpu.VMEM((CHUNK, d), jnp.float32),
            pltpu.SemaphoreType.DMA,
            pltpu.SemaphoreType.DMA,
            pltpu.SemaphoreType.DMA,
            pltpu.SemaphoreType.DMA,
        ],
    )(idx, position_table, token_table)
    return out.reshape(bsz, seq, d)


# speedup vs baseline: 1.0096x; 1.0096x over previous
import functools

import jax
import jax.numpy as jnp
from jax import lax
from jax.experimental import pallas as pl
from jax.experimental.pallas import tpu as pltpu
from jax.experimental.pallas import tpu_sc as plsc

D = 64
SEQ = 200
L = 16
NC = 2
NS = 16
NW = NC * NS

CHUNK = 40
NCH = SEQ // CHUNK
SCALE = 8.0


def _fire_gathers(tab_hbm, idx_v, bank, sem, s_local):
    for c in range(NCH):
        pltpu.async_copy(
            tab_hbm.at[idx_v.at[s_local, pl.ds(c * CHUNK, CHUNK)]],
            bank.at[pl.ds(c * CHUNK, CHUNK)],
            sem,
        )


def _sc_body(idx_hbm, pos_hbm, tab_hbm, out_hbm,
             idx_v, pos_v, bank0, bank1, sem0, sem1, *, seq_per_w):
    wid = lax.axis_index("s") * NC + lax.axis_index("c")
    seq0 = wid * seq_per_w
    pltpu.sync_copy(idx_hbm.at[pl.ds(seq0, seq_per_w)], idx_v)
    pltpu.sync_copy(pos_hbm, pos_v)

    _fire_gathers(tab_hbm, idx_v, bank0, sem0, 0)
    _fire_gathers(tab_hbm, idx_v, bank1, sem1, 1)

    banks = (bank0, bank1)
    sems = (sem0, sem1)

    def seq_step(s_local, bank, sem):
        pltpu.make_async_copy(tab_hbm.at[pl.ds(0, SEQ)], bank, sem).wait()

        def ew(rr, carry):
            r = rr * 4
            for j in range(4):
                for c in range(D // L):
                    sl = pl.ds(c * L, L)
                    bank[r + j, sl] = bank[r + j, sl] * SCALE + pos_v[r + j, sl]
            return carry
        lax.fori_loop(0, SEQ // 4, ew, 0, unroll=False)

        pltpu.sync_copy(bank, out_hbm.at[seq0 + s_local])

        @pl.when(s_local + 2 < seq_per_w)
        def _():
            _fire_gathers(tab_hbm, idx_v, bank, sem, s_local + 2)

    def outer(it, carry):
        s0 = it * 2
        for half in range(2):
            seq_step(s0 + half, banks[half], sems[half])
        return carry

    lax.fori_loop(0, seq_per_w // 2, outer, 0, unroll=False)


def kernel(inputs, token_table, position_table):
    bsz, seq = inputs.shape
    vocab, d = token_table.shape
    seq_per_w = bsz // NW

    mesh = plsc.VectorSubcoreMesh(core_axis_name="c", subcore_axis_name="s")
    body = functools.partial(_sc_body, seq_per_w=seq_per_w)

    return pl.kernel(
        body,
        out_type=jax.ShapeDtypeStruct((bsz, seq, d), jnp.float32),
        mesh=mesh,
        compiler_params=pltpu.CompilerParams(use_tc_tiling_on_sc=False),
        scratch_types=[
            pltpu.VMEM((seq_per_w, seq), jnp.int32),
            pltpu.VMEM((seq, d), jnp.float32),
            pltpu.VMEM((seq, d), jnp.float32),
            pltpu.VMEM((seq, d), jnp.float32),
            pltpu.SemaphoreType.DMA,
            pltpu.SemaphoreType.DMA,
        ],
    )(inputs.astype(jnp.int32), position_table, token_table)
